# Initial kernel scaffold; baseline (speedup 1.0000x reference)
import jax
import jax.numpy as jnp
from jax.experimental import pallas as pl


def kernel(q, keys, values):
    raise NotImplementedError("write your pallas kernel here")


# speedup vs baseline: 4.2416x; 4.2416x over previous
import functools

import jax
import jax.numpy as jnp
from jax import lax
from jax.experimental import pallas as pl
from jax.experimental.pallas import tpu as pltpu
from jax.experimental.pallas import tpu_sc as plsc

NEG = -1e30
BIGI = 1 << 30
CH = 128
TOPK = 8


def _k1_body(nblk, bq, bk, dk, q_ref, k_ref, sims_ref, c_ref, qn_ref):
    j = pl.program_id(0)

    @pl.when(j == 0)
    def _():
        qv = q_ref[...]
        nrm = jnp.sqrt(jnp.sum(qv * qv, axis=1, keepdims=True))
        qn_ref[...] = qv / (nrm + 1e-8)

    kv = k_ref[...]
    knrm = jnp.sqrt(jnp.sum(kv * kv, axis=1, keepdims=True))
    kn = kv / (knrm + 1e-8)
    sims = lax.dot_general(qn_ref[...], kn,
                           dimension_numbers=(((1,), (1,)), ((), ())),
                           preferred_element_type=jnp.float32)
    sims_ref[...] = sims
    cpb = bk // CH
    cm = [jnp.max(sims[:, c * CH:(c + 1) * CH], axis=1, keepdims=True)
          for c in range(cpb)]
    c_ref[...] = jnp.concatenate(cm, axis=1)[None]


def _run_k1(q, keys):
    bq, dk = q.shape
    n = keys.shape[0]
    bk = 2048
    nblk = n // bk
    return pl.pallas_call(
        functools.partial(_k1_body, nblk, bq, bk, dk),
        grid=(nblk,),
        in_specs=[
            pl.BlockSpec((bq, dk), lambda j: (0, 0)),
            pl.BlockSpec((bk, dk), lambda j: (j, 0)),
        ],
        out_specs=[
            pl.BlockSpec((bq, bk), lambda j: (0, j)),
            pl.BlockSpec((1, bq, bk // CH), lambda j: (j, 0, 0)),
        ],
        out_shape=[
            jax.ShapeDtypeStruct((bq, n), jnp.float32),
            jax.ShapeDtypeStruct((nblk, bq, bk // CH), jnp.float32),
        ],
        scratch_shapes=[pltpu.VMEM((bq, dk), jnp.float32)],
        compiler_params=pltpu.CompilerParams(
            dimension_semantics=("arbitrary",),
            vmem_limit_bytes=100 * 1024 * 1024,
        ),
    )(q, keys)


def _chunk_top8_body(nchunk, bq, c_ref, out_ref):
    vals = c_ref[...]
    iota = lax.broadcasted_iota(jnp.int32, (bq, nchunk), 1)
    sels = []
    for _ in range(TOPK):
        m = jnp.max(vals, axis=1, keepdims=True)
        eq = vals == m
        sel = jnp.min(jnp.where(eq, iota, BIGI), axis=1, keepdims=True)
        sels.append(sel)
        vals = jnp.where(iota == sel, NEG, vals)
    pad = jnp.zeros((bq, CH - TOPK), jnp.int32)
    out_ref[...] = jnp.concatenate(sels + [pad], axis=1)


def _run_chunk_top8(c):
    bq, nchunk = c.shape
    return pl.pallas_call(
        functools.partial(_chunk_top8_body, nchunk, bq),
        out_shape=jax.ShapeDtypeStruct((bq, CH), jnp.int32),
        compiler_params=pltpu.CompilerParams(
            vmem_limit_bytes=100 * 1024 * 1024),
    )(c)


def _final_body(bq, ncand, cands_ref, chunk_ref, idx_ref, attn_ref, conf_ref):
    lane = lax.broadcasted_iota(jnp.int32, (bq, CH), 1)
    kidx = jnp.concatenate(
        [chunk_ref[:, s:s + 1] * CH + lane for s in range(TOPK)], axis=1)
    vals = cands_ref[...]
    tvs, tis = [], []
    for _ in range(TOPK):
        m = jnp.max(vals, axis=1, keepdims=True)
        eq = vals == m
        sel = jnp.min(jnp.where(eq, kidx, BIGI), axis=1, keepdims=True)
        tvs.append(m)
        tis.append(sel)
        vals = jnp.where(kidx == sel, NEG, vals)
    tv = jnp.concatenate(tvs, axis=1)
    ti = jnp.concatenate(tis, axis=1)
    logits = tv / jnp.float32(0.1)
    mx = jnp.max(logits, axis=1, keepdims=True)
    e = jnp.exp(logits - mx)
    attn = e / jnp.sum(e, axis=1, keepdims=True)
    p = jnp.maximum(attn, 1e-8)
    h = -jnp.sum(p * jnp.log(p), axis=1, keepdims=True)
    hmax = jnp.log(jnp.float32(TOPK))
    conf = jnp.sum(1.0 - h / (hmax + 1e-8)) / bq
    zpad_i = jnp.zeros((bq, CH - TOPK), jnp.int32)
    zpad_f = jnp.zeros((bq, CH - TOPK), jnp.float32)
    idx_ref[...] = jnp.concatenate([ti, zpad_i], axis=1)
    attn_ref[...] = jnp.concatenate([attn, zpad_f], axis=1)
    conf_ref[...] = jnp.full((8, CH), conf, jnp.float32)


def _run_final(cands, chunk_pad):
    bq, ncand = cands.shape
    return pl.pallas_call(
        functools.partial(_final_body, bq, ncand),
        out_shape=[
            jax.ShapeDtypeStruct((bq, CH), jnp.int32),
            jax.ShapeDtypeStruct((bq, CH), jnp.float32),
            jax.ShapeDtypeStruct((8, CH), jnp.float32),
        ],
        compiler_params=pltpu.CompilerParams(
            vmem_limit_bytes=100 * 1024 * 1024),
    )(cands, chunk_pad)


def _combine_body(bq, dv, rows_ref, attn_ref, out_ref):
    acc = attn_ref[:, 0:1] * rows_ref[0]
    for s in range(1, TOPK):
        acc = acc + attn_ref[:, s:s + 1] * rows_ref[s]
    out_ref[...] = acc


def _run_combine(rows_t, attn_pad):
    _, bq, dv = rows_t.shape
    return pl.pallas_call(
        functools.partial(_combine_body, bq, dv),
        out_shape=jax.ShapeDtypeStruct((bq, dv), jnp.float32),
        compiler_params=pltpu.CompilerParams(
            vmem_limit_bytes=100 * 1024 * 1024),
    )(rows_t, attn_pad)


def _sc_gather(table, idx2d):
    v, d = table.shape
    b = idx2d.shape[0] * 128
    info = plsc.get_sparse_core_info()
    nc, ns = info.num_cores, info.num_subcores
    nw = nc * ns
    b_per_w = b // nw
    irows = b_per_w // 128
    mesh = plsc.VectorSubcoreMesh(core_axis_name="c", subcore_axis_name="s")

    @functools.partial(
        pl.kernel, mesh=mesh,
        out_type=jax.ShapeDtypeStruct((b, d), jnp.float32),
        scratch_types=[
            pltpu.VMEM((irows, 128), jnp.int32),
            pltpu.VMEM((b_per_w, d), jnp.float32),
            pltpu.SemaphoreType.DMA,
        ],
    )
    def k(table_hbm, idx_hbm, out_hbm, idx_v, rows_v, sem):
        wid = lax.axis_index("s") * nc + lax.axis_index("c")
        pltpu.sync_copy(idx_hbm.at[pl.ds(wid * irows, irows)], idx_v)
        copies = [
            pltpu.async_copy(table_hbm.at[idx_v.at[r]],
                             rows_v.at[pl.ds(r * 128, 128)], sem)
            for r in range(irows)
        ]
        for cp in copies:
            cp.wait()
        pltpu.sync_copy(rows_v, out_hbm.at[pl.ds(wid * b_per_w, b_per_w)])

    return k(table, idx2d)


def kernel(q, keys, values):
    bq, dk = q.shape
    n, dv = values.shape
    nchunk = n // CH

    sims, c3 = _run_k1(q, keys)
    c = jnp.transpose(c3, (1, 0, 2)).reshape(bq, nchunk)
    chunk_pad = _run_chunk_top8(c)
    chunk8 = chunk_pad[:, :TOPK]

    gidx = (jnp.arange(bq, dtype=jnp.int32)[:, None] * nchunk
            + chunk8).reshape(bq * TOPK // 128, 128)
    cands = _sc_gather(sims.reshape(bq * nchunk, CH), gidx)
    cands = cands.reshape(bq, TOPK * CH)

    idx_pad, attn_pad, conf_tile = _run_final(cands, chunk_pad)
    top_idx = idx_pad[:, :TOPK]
    attn = attn_pad[:, :TOPK]

    rows = _sc_gather(values, top_idx.reshape(bq * TOPK // 128, 128))
    rows_t = jnp.transpose(rows.reshape(bq, TOPK, dv), (1, 0, 2))
    out = _run_combine(rows_t, attn_pad)
    return (out, conf_tile[0, 0], top_idx, attn)


# speedup vs baseline: 9.6737x; 2.2807x over previous
import functools

import jax
import jax.numpy as jnp
from jax import lax
from jax.experimental import pallas as pl
from jax.experimental.pallas import tpu as pltpu
from jax.experimental.pallas import tpu_sc as plsc

NEG = -1e30
BIGI = 1 << 30
CH = 128
TOPK = 8


def _k1_body(nblk, bq, bk, dk, q_ref, k_ref, sims_ref, c_ref, qn_ref):
    j = pl.program_id(0)

    @pl.when(j == 0)
    def _():
        qv = q_ref[...]
        nrm = jnp.sqrt(jnp.sum(qv * qv, axis=1, keepdims=True))
        qn_ref[...] = qv / (nrm + 1e-8)

    kv = k_ref[...]
    knrm = jnp.sqrt(jnp.sum(kv * kv, axis=1, keepdims=True))
    kn = kv / (knrm + 1e-8)
    sims = lax.dot_general(qn_ref[...], kn,
                           dimension_numbers=(((1,), (1,)), ((), ())),
                           preferred_element_type=jnp.float32)
    cpb = bk // CH
    cm = []
    for c in range(cpb):
        blk = sims[:, c * CH:(c + 1) * CH]
        sims_ref[c] = blk
        cm.append(jnp.max(blk, axis=1, keepdims=True))
    c_ref[...] = jnp.concatenate(cm, axis=1)[None]


def _run_k1(q, keys):
    bq, dk = q.shape
    n = keys.shape[0]
    bk = 2048
    nblk = n // bk
    return pl.pallas_call(
        functools.partial(_k1_body, nblk, bq, bk, dk),
        grid=(nblk,),
        in_specs=[
            pl.BlockSpec((bq, dk), lambda j: (0, 0)),
            pl.BlockSpec((bk, dk), lambda j: (j, 0)),
        ],
        out_specs=[
            pl.BlockSpec((bk // CH, bq, CH), lambda j: (j, 0, 0)),
            pl.BlockSpec((1, bq, bk // CH), lambda j: (j, 0, 0)),
        ],
        out_shape=[
            jax.ShapeDtypeStruct((n // CH, bq, CH), jnp.float32),
            jax.ShapeDtypeStruct((nblk, bq, bk // CH), jnp.float32),
        ],
        scratch_shapes=[pltpu.VMEM((bq, dk), jnp.float32)],
        compiler_params=pltpu.CompilerParams(
            dimension_semantics=("arbitrary",),
            vmem_limit_bytes=100 * 1024 * 1024,
        ),
    )(q, keys)


def _chunk_top8_body(nchunk, bq, c_ref, out_ref):
    vals = c_ref[...]
    iota = lax.broadcasted_iota(jnp.int32, (bq, nchunk), 1)
    sels = []
    for _ in range(TOPK):
        m = jnp.max(vals, axis=1, keepdims=True)
        eq = vals == m
        sel = jnp.min(jnp.where(eq, iota, BIGI), axis=1, keepdims=True)
        sels.append(sel)
        vals = jnp.where(iota == sel, NEG, vals)
    pad = jnp.zeros((bq, CH - TOPK), jnp.int32)
    out_ref[...] = jnp.concatenate(sels + [pad], axis=1)


def _run_chunk_top8(c):
    bq, nchunk = c.shape
    return pl.pallas_call(
        functools.partial(_chunk_top8_body, nchunk, bq),
        out_shape=jax.ShapeDtypeStruct((bq, CH), jnp.int32),
        compiler_params=pltpu.CompilerParams(
            vmem_limit_bytes=100 * 1024 * 1024),
    )(c)


def _final_body(bq, ncand, cands_ref, chunk_ref, idx_ref, attn_ref, conf_ref):
    lane = lax.broadcasted_iota(jnp.int32, (bq, CH), 1)
    kidx = jnp.concatenate(
        [chunk_ref[:, s:s + 1] * CH + lane for s in range(TOPK)], axis=1)
    c3 = cands_ref[...]
    vals = jnp.concatenate([c3[:, s, :] for s in range(TOPK)], axis=1)
    tvs, tis = [], []
    for _ in range(TOPK):
        m = jnp.max(vals, axis=1, keepdims=True)
        eq = vals == m
        sel = jnp.min(jnp.where(eq, kidx, BIGI), axis=1, keepdims=True)
        tvs.append(m)
        tis.append(sel)
        vals = jnp.where(kidx == sel, NEG, vals)
    tv = jnp.concatenate(tvs, axis=1)
    ti = jnp.concatenate(tis, axis=1)
    logits = tv / jnp.float32(0.1)
    mx = jnp.max(logits, axis=1, keepdims=True)
    e = jnp.exp(logits - mx)
    attn = e / jnp.sum(e, axis=1, keepdims=True)
    p = jnp.maximum(attn, 1e-8)
    h = -jnp.sum(p * jnp.log(p), axis=1, keepdims=True)
    hmax = jnp.log(jnp.float32(TOPK))
    conf = jnp.sum(1.0 - h / (hmax + 1e-8)) / bq
    zpad_i = jnp.zeros((bq, CH - TOPK), jnp.int32)
    zpad_f = jnp.zeros((bq, CH - TOPK), jnp.float32)
    idx_ref[...] = jnp.concatenate([ti, zpad_i], axis=1)
    attn_ref[...] = jnp.concatenate([attn, zpad_f], axis=1)
    conf_ref[...] = jnp.full((8, CH), conf, jnp.float32)


def _run_final(cands, chunk_pad):
    bq = cands.shape[0]
    ncand = TOPK * CH
    return pl.pallas_call(
        functools.partial(_final_body, bq, ncand),
        out_shape=[
            jax.ShapeDtypeStruct((bq, CH), jnp.int32),
            jax.ShapeDtypeStruct((bq, CH), jnp.float32),
            jax.ShapeDtypeStruct((8, CH), jnp.float32),
        ],
        compiler_params=pltpu.CompilerParams(
            vmem_limit_bytes=100 * 1024 * 1024),
    )(cands, chunk_pad)


def _combine_body(bq, dv, rows_ref, attn_ref, out_ref):
    rows = rows_ref[...]
    acc = attn_ref[:, 0:1] * rows[:, 0, :]
    for s in range(1, TOPK):
        acc = acc + attn_ref[:, s:s + 1] * rows[:, s, :]
    out_ref[...] = acc


def _run_combine(rows_t, attn_pad):
    bq, _, dv = rows_t.shape
    return pl.pallas_call(
        functools.partial(_combine_body, bq, dv),
        out_shape=jax.ShapeDtypeStruct((bq, dv), jnp.float32),
        compiler_params=pltpu.CompilerParams(
            vmem_limit_bytes=100 * 1024 * 1024),
    )(rows_t, attn_pad)


def _sc_gather(table, idx2d):
    v, d = table.shape
    b = idx2d.shape[0] * 128
    info = plsc.get_sparse_core_info()
    nc, ns = info.num_cores, info.num_subcores
    nw = nc * ns
    b_per_w = b // nw
    irows = b_per_w // 128
    mesh = plsc.VectorSubcoreMesh(core_axis_name="c", subcore_axis_name="s")

    @functools.partial(
        pl.kernel, mesh=mesh,
        out_type=jax.ShapeDtypeStruct((b, d), jnp.float32),
        scratch_types=[
            pltpu.VMEM((irows, 128), jnp.int32),
            pltpu.VMEM((b_per_w, d), jnp.float32),
            pltpu.SemaphoreType.DMA,
        ],
    )
    def k(table_hbm, idx_hbm, out_hbm, idx_v, rows_v, sem):
        wid = lax.axis_index("s") * nc + lax.axis_index("c")
        pltpu.sync_copy(idx_hbm.at[pl.ds(wid * irows, irows)], idx_v)
        copies = [
            pltpu.async_copy(table_hbm.at[idx_v.at[r]],
                             rows_v.at[pl.ds(r * 128, 128)], sem)
            for r in range(irows)
        ]
        for cp in copies:
            cp.wait()
        pltpu.sync_copy(rows_v, out_hbm.at[pl.ds(wid * b_per_w, b_per_w)])

    return k(table, idx2d)


def kernel(q, keys, values):
    bq, dk = q.shape
    n, dv = values.shape
    nchunk = n // CH

    sims3, cb3 = _run_k1(q, keys)
    c = jnp.transpose(cb3, (1, 0, 2)).reshape(bq, nchunk)
    chunk_pad = _run_chunk_top8(c)
    chunk8 = chunk_pad[:, :TOPK]

    gidx = (chunk8 * bq
            + jnp.arange(bq, dtype=jnp.int32)[:, None]
            ).reshape(bq * TOPK // 128, 128)
    cands = _sc_gather(sims3.reshape(nchunk * bq, CH), gidx)
    cands = cands.reshape(bq, TOPK, CH)

    idx_pad, attn_pad, conf_tile = _run_final(cands, chunk_pad)
    top_idx = idx_pad[:, :TOPK]
    attn = attn_pad[:, :TOPK]

    rows = _sc_gather(values, top_idx.reshape(bq * TOPK // 128, 128))
    out = _run_combine(rows.reshape(bq, TOPK, dv), attn_pad)
    return (out, conf_tile[0, 0], top_idx, attn)
